# baseline (speedup 1.0000x reference)
import jax
import jax.numpy as jnp
from jax import lax
from jax.experimental import pallas as pl
from jax.experimental.pallas import tpu as pltpu
from jax.experimental.pallas import tpu_sc as plsc

_NC = 2
_NS = 16
_K = 128
_EPS = 1e-5


def _sc_scatter(y, ei, zeros):
    N, D = y.shape
    NCH = ei.shape[0] // (_NC * _NS * 2 * _K)
    Np = zeros.shape[0]
    RB = Np // _NS

    mesh = plsc.VectorSubcoreMesh(core_axis_name="c", subcore_axis_name="s")
    out_type = jax.ShapeDtypeStruct((_NC, Np, D), jnp.float32)
    scratch = [
        pltpu.VMEM((2, 2 * _K), jnp.int32),
        pltpu.VMEM((_K,), jnp.int32),
        pltpu.VMEM((2, _K, D), jnp.float32),
        pltpu.VMEM_SHARED((Np, D), jnp.float32),
        pltpu.SemaphoreType.DMA((2,)),
        pltpu.SemaphoreType.DMA((2,)),
    ]

    def body(y_hbm, ei_hbm, z_hbm, sout, land, dsc, rows, accum, isem, gsem):
        cid = lax.axis_index("c")
        sid = lax.axis_index("s")
        widx = cid * _NS + sid
        r0 = sid * RB
        last = NCH - 1

        def issue_idx(g):
            gc = jnp.minimum(g, last)
            e0 = (widx * NCH + gc) * (2 * _K)
            b = lax.rem(g, 2)
            pltpu.async_copy(ei_hbm.at[pl.ds(e0, 2 * _K)], land.at[b],
                             isem.at[b])

        def wait_idx(g):
            b = lax.rem(g, 2)
            pltpu.make_async_copy(ei_hbm.at[pl.ds(0, 2 * _K)], land.at[b],
                                  isem.at[b]).wait()

        def issue_gather(g):
            b = lax.rem(g, 2)
            pltpu.async_copy(y_hbm.at[land.at[b, pl.ds(0, _K)]], rows.at[b],
                             gsem.at[b])

        def wait_gather(g):
            b = lax.rem(g, 2)
            pltpu.make_async_copy(y_hbm.at[land.at[b, pl.ds(0, _K)]],
                                  rows.at[b], gsem.at[b]).wait()

        pltpu.sync_copy(z_hbm.at[pl.ds(r0, RB), :], accum.at[pl.ds(r0, RB), :])
        plsc.subcore_barrier()

        issue_idx(0)
        wait_idx(0)
        issue_gather(0)
        issue_idx(1)

        def step(g, carry):
            b = lax.rem(g, 2)
            wait_idx(g + 1)
            issue_gather(g + 1)
            for j in range(_K // 16):
                dsc[pl.ds(j * 16, 16)] = land[b, pl.ds(_K + j * 16, 16)]
            wait_gather(g)
            pltpu.sync_copy(rows.at[b], accum.at[dsc], add=True)
            issue_idx(g + 2)
            return carry
        lax.fori_loop(0, NCH, step, 0)

        wait_idx(NCH + 1)
        wait_gather(NCH)

        plsc.subcore_barrier()
        pltpu.sync_copy(accum.at[pl.ds(r0, RB), :],
                        sout.at[cid, pl.ds(r0, RB), :])

    k = pl.kernel(body, out_type=out_type, mesh=mesh, scratch_types=scratch)
    return k(y, ei, zeros)


def _sc_degree(ei, zeros, ones):
    NCH = ei.shape[0] // (_NC * _NS * 2 * _K)
    Np, D = zeros.shape
    RB = Np // _NS

    mesh = plsc.VectorSubcoreMesh(core_axis_name="c", subcore_axis_name="s")
    out_type = jax.ShapeDtypeStruct((_NC, Np, D), jnp.float32)
    scratch = [
        pltpu.VMEM((2, _K), jnp.int32),
        pltpu.VMEM((_K,), jnp.int32),
        pltpu.VMEM((_K,), jnp.int32),
        pltpu.VMEM((_K, D), jnp.float32),
        pltpu.VMEM_SHARED((Np, D), jnp.float32),
        pltpu.SemaphoreType.DMA((2,)),
        pltpu.SemaphoreType.DMA((2,)),
    ]

    def body(dst_hbm, z_hbm, ones_hbm, dout, dstv, dsc0, dsc1, onesv, dega,
             isem, ssem):
        cid = lax.axis_index("c")
        sid = lax.axis_index("s")
        widx = cid * _NS + sid
        r0 = sid * RB
        last = NCH - 1

        def issue_idx(g):
            gc = jnp.minimum(g, last)
            e0 = (widx * NCH + gc) * (2 * _K) + _K
            b = lax.rem(g, 2)
            pltpu.async_copy(dst_hbm.at[pl.ds(e0, _K)], dstv.at[b],
                             isem.at[b])

        def wait_idx(g):
            b = lax.rem(g, 2)
            pltpu.make_async_copy(dst_hbm.at[pl.ds(0, _K)], dstv.at[b],
                                  isem.at[b]).wait()

        def issue_scatter(g, dscX):
            b = lax.rem(g, 2)
            for j in range(_K // 16):
                dscX[pl.ds(j * 16, 16)] = dstv[b, pl.ds(j * 16, 16)]
            pltpu.async_copy(onesv, dega.at[dscX], ssem.at[b], add=True)

        def wait_scatter(g):
            b = lax.rem(g, 2)
            pltpu.make_async_copy(onesv, dega.at[dsc0], ssem.at[b]).wait()

        pltpu.sync_copy(z_hbm.at[pl.ds(r0, RB), :], dega.at[pl.ds(r0, RB), :])
        pltpu.sync_copy(ones_hbm, onesv)
        plsc.subcore_barrier()

        issue_idx(0)

        def step(g, carry):
            b = lax.rem(g, 2)
            wait_idx(g)
            issue_idx(g + 1)

            @pl.when(g >= 2)
            def _():
                wait_scatter(g - 2)

            @pl.when(b == 0)
            def _():
                issue_scatter(g, dsc0)

            @pl.when(b == 1)
            def _():
                issue_scatter(g, dsc1)
            return carry
        lax.fori_loop(0, NCH, step, 0)

        wait_idx(NCH)
        wait_scatter(NCH - 2)
        wait_scatter(NCH - 1)

        plsc.subcore_barrier()
        pltpu.sync_copy(dega.at[pl.ds(r0, RB), :],
                        dout.at[cid, pl.ds(r0, RB), :])

    k = pl.kernel(body, out_type=out_type, mesh=mesh, scratch_types=scratch)
    return k(ei, zeros, ones)


def _dotT(a, w):
    return lax.dot_general(a, w, (((1,), (1,)), ((), ())),
                           preferred_element_type=jnp.float32)


def _pre_body(x_ref, wl_ref, wr_ref, b_ref, y_ref, z_ref):
    x = x_ref[...]
    y_ref[...] = _dotT(x, wl_ref[...])
    z_ref[...] = _dotT(x, wr_ref[...]) + b_ref[...]


def _bn_relu(s_ref, degp_ref, z_ref, g_ref, be_ref):
    n = z_ref.shape[0]
    s = (s_ref[0] + s_ref[1])[:n]
    deg = (degp_ref[0] + degp_ref[1])[:n]
    h = s / jnp.maximum(deg, 1.0) + z_ref[...]
    mu = jnp.mean(h, axis=0, keepdims=True)
    ctr = h - mu
    var = jnp.mean(ctr * ctr, axis=0, keepdims=True)
    hn = g_ref[...] * ctr * lax.rsqrt(var + _EPS) + be_ref[...]
    return jnp.maximum(hn, 0.0)


def _mid_body(s_ref, degp_ref, z_ref, g_ref, be_ref, wl_ref, wr_ref,
              b_ref, y2_ref, z2_ref):
    h1 = _bn_relu(s_ref, degp_ref, z_ref, g_ref, be_ref)
    y2_ref[...] = _dotT(h1, wl_ref[...])
    z2_ref[...] = _dotT(h1, wr_ref[...]) + b_ref[...]


def _post_body(s_ref, degp_ref, z_ref, g_ref, be_ref, out_ref):
    out_ref[...] = _bn_relu(s_ref, degp_ref, z_ref, g_ref, be_ref)


def kernel(x, edge_index, Wl1, Wr1, b1, g1, be1, Wl2, Wr2, b2, g2, be2):
    N, D = x.shape
    E = edge_index.shape[1]
    Np = (N + 16 * 8 - 1) // (16 * 8) * (16 * 8)
    NW = _NC * _NS
    ept = (E // NW + _K - 1) // _K * _K
    pad = NW * ept - E
    src1 = jnp.concatenate([edge_index[0], jnp.zeros((pad,), jnp.int32)])
    dst1 = jnp.concatenate([edge_index[1],
                            jnp.full((pad,), Np - 1, jnp.int32)])
    ei = jnp.stack([src1.reshape(-1, _K), dst1.reshape(-1, _K)],
                   axis=1).reshape(-1)
    zeros = jnp.zeros((Np, D), jnp.float32)
    ones = jnp.ones((_K, D), jnp.float32)
    f32 = jnp.float32
    sd = jax.ShapeDtypeStruct

    degp = _sc_degree(ei, zeros, ones)

    y1, z1 = pl.pallas_call(
        _pre_body,
        out_shape=[sd((N, D), f32), sd((N, D), f32)],
    )(x, Wl1, Wr1, b1.reshape(1, D))

    s1 = _sc_scatter(y1, ei, zeros)

    y2, z2 = pl.pallas_call(
        _mid_body,
        out_shape=[sd((N, D), f32), sd((N, D), f32)],
    )(s1, degp, z1, g1.reshape(1, D), be1.reshape(1, D), Wl2, Wr2,
      b2.reshape(1, D))

    s2 = _sc_scatter(y2, ei, zeros)

    out = pl.pallas_call(
        _post_body,
        out_shape=sd((N, D), f32),
    )(s2, degp, z2, g2.reshape(1, D), be2.reshape(1, D))

    return out


# speedup vs baseline: 1.8371x; 1.8371x over previous
import jax
import jax.numpy as jnp
from jax import lax
from jax.experimental import pallas as pl
from jax.experimental.pallas import tpu as pltpu
from jax.experimental.pallas import tpu_sc as plsc

_NC = 2
_NS = 16
_K = 128
_EPS = 1e-5


def _sc_scatter(y, ei, zeros):
    N, D = y.shape
    NCH = ei.shape[0] // (_NC * _NS * 2 * _K)
    Np = zeros.shape[0]
    RB = Np // _NS

    mesh = plsc.VectorSubcoreMesh(core_axis_name="c", subcore_axis_name="s")
    out_type = jax.ShapeDtypeStruct((_NC, Np, D), jnp.float32)
    scratch = [
        pltpu.VMEM((2, 2 * _K), jnp.int32),
        pltpu.VMEM((_K,), jnp.int32),
        pltpu.VMEM((2, _K, D), jnp.float32),
        pltpu.VMEM_SHARED((Np, D), jnp.float32),
        pltpu.SemaphoreType.DMA((2,)),
        pltpu.SemaphoreType.DMA((2,)),
    ]

    def body(y_hbm, ei_hbm, z_hbm, sout, land, dsc, rows, accum, isem, gsem):
        cid = lax.axis_index("c")
        sid = lax.axis_index("s")
        widx = cid * _NS + sid
        r0 = sid * RB
        last = NCH - 1

        def issue_idx(g):
            gc = jnp.minimum(g, last)
            e0 = (widx * NCH + gc) * (2 * _K)
            b = lax.rem(g, 2)
            pltpu.async_copy(ei_hbm.at[pl.ds(e0, 2 * _K)], land.at[b],
                             isem.at[b])

        def wait_idx(g):
            b = lax.rem(g, 2)
            pltpu.make_async_copy(ei_hbm.at[pl.ds(0, 2 * _K)], land.at[b],
                                  isem.at[b]).wait()

        def issue_gather(g):
            b = lax.rem(g, 2)
            pltpu.async_copy(y_hbm.at[land.at[b, pl.ds(0, _K)]], rows.at[b],
                             gsem.at[b])

        def wait_gather(g):
            b = lax.rem(g, 2)
            pltpu.make_async_copy(y_hbm.at[land.at[b, pl.ds(0, _K)]],
                                  rows.at[b], gsem.at[b]).wait()

        pltpu.sync_copy(z_hbm.at[pl.ds(r0, RB), :], accum.at[pl.ds(r0, RB), :])
        plsc.subcore_barrier()

        issue_idx(0)
        wait_idx(0)
        issue_gather(0)
        issue_idx(1)

        def step(g, carry):
            b = lax.rem(g, 2)
            wait_idx(g + 1)
            issue_gather(g + 1)
            for j in range(_K // 16):
                dsc[pl.ds(j * 16, 16)] = land[b, pl.ds(_K + j * 16, 16)]
            wait_gather(g)
            pltpu.sync_copy(rows.at[b], accum.at[dsc], add=True)
            issue_idx(g + 2)
            return carry
        lax.fori_loop(0, NCH, step, 0)

        wait_idx(NCH + 1)
        wait_gather(NCH)

        plsc.subcore_barrier()
        pltpu.sync_copy(accum.at[pl.ds(r0, RB), :],
                        sout.at[cid, pl.ds(r0, RB), :])

    k = pl.kernel(body, out_type=out_type, mesh=mesh, scratch_types=scratch)
    return k(y, ei, zeros)


def _sc_degree(ei, zeros, ones):
    NCH = ei.shape[0] // (_NC * _NS * 2 * _K)
    Np, D = zeros.shape
    RB = Np // _NS

    mesh = plsc.VectorSubcoreMesh(core_axis_name="c", subcore_axis_name="s")
    out_type = jax.ShapeDtypeStruct((_NC, Np, D), jnp.float32)
    scratch = [
        pltpu.VMEM((2, _K), jnp.int32),
        pltpu.VMEM((_K,), jnp.int32),
        pltpu.VMEM((_K,), jnp.int32),
        pltpu.VMEM((_K, D), jnp.float32),
        pltpu.VMEM_SHARED((Np, D), jnp.float32),
        pltpu.SemaphoreType.DMA((2,)),
        pltpu.SemaphoreType.DMA((2,)),
    ]

    def body(dst_hbm, z_hbm, ones_hbm, dout, dstv, dsc0, dsc1, onesv, dega,
             isem, ssem):
        cid = lax.axis_index("c")
        sid = lax.axis_index("s")
        widx = cid * _NS + sid
        r0 = sid * RB
        last = NCH - 1

        def issue_idx(g):
            gc = jnp.minimum(g, last)
            e0 = (widx * NCH + gc) * (2 * _K) + _K
            b = lax.rem(g, 2)
            pltpu.async_copy(dst_hbm.at[pl.ds(e0, _K)], dstv.at[b],
                             isem.at[b])

        def wait_idx(g):
            b = lax.rem(g, 2)
            pltpu.make_async_copy(dst_hbm.at[pl.ds(0, _K)], dstv.at[b],
                                  isem.at[b]).wait()

        def issue_scatter(g, dscX):
            b = lax.rem(g, 2)
            for j in range(_K // 16):
                dscX[pl.ds(j * 16, 16)] = dstv[b, pl.ds(j * 16, 16)]
            pltpu.async_copy(onesv, dega.at[dscX], ssem.at[b], add=True)

        def wait_scatter(g):
            b = lax.rem(g, 2)
            pltpu.make_async_copy(onesv, dega.at[dsc0], ssem.at[b]).wait()

        pltpu.sync_copy(z_hbm.at[pl.ds(r0, RB), :], dega.at[pl.ds(r0, RB), :])
        pltpu.sync_copy(ones_hbm, onesv)
        plsc.subcore_barrier()

        issue_idx(0)

        def step(g, carry):
            b = lax.rem(g, 2)
            wait_idx(g)
            issue_idx(g + 1)

            @pl.when(g >= 2)
            def _():
                wait_scatter(g - 2)

            @pl.when(b == 0)
            def _():
                issue_scatter(g, dsc0)

            @pl.when(b == 1)
            def _():
                issue_scatter(g, dsc1)
            return carry
        lax.fori_loop(0, NCH, step, 0)

        wait_idx(NCH)
        wait_scatter(NCH - 2)
        wait_scatter(NCH - 1)

        plsc.subcore_barrier()
        pltpu.sync_copy(dega.at[pl.ds(r0, RB), :],
                        dout.at[cid, pl.ds(r0, RB), :])

    k = pl.kernel(body, out_type=out_type, mesh=mesh, scratch_types=scratch)
    return k(ei, zeros, ones)


def _dotT(a, w):
    return lax.dot_general(a, w, (((1,), (1,)), ((), ())),
                           preferred_element_type=jnp.float32)


def _pre_body(x_ref, wl_ref, wr_ref, b_ref, y_ref, z_ref):
    x = x_ref[...]
    y_ref[...] = _dotT(x, wl_ref[...])
    z_ref[...] = _dotT(x, wr_ref[...]) + b_ref[...]


def _bn_relu(s_ref, degp_ref, z_ref, g_ref, be_ref):
    n = z_ref.shape[0]
    s = (s_ref[0] + s_ref[1])[:n]
    deg = (degp_ref[0] + degp_ref[1])[:n]
    h = s / jnp.maximum(deg, 1.0) + z_ref[...]
    mu = jnp.mean(h, axis=0, keepdims=True)
    ctr = h - mu
    var = jnp.mean(ctr * ctr, axis=0, keepdims=True)
    hn = g_ref[...] * ctr * lax.rsqrt(var + _EPS) + be_ref[...]
    return jnp.maximum(hn, 0.0)


def _mid_body(s_ref, degp_ref, z_ref, g_ref, be_ref, wl_ref, wr_ref,
              b_ref, y2_ref, z2_ref):
    h1 = _bn_relu(s_ref, degp_ref, z_ref, g_ref, be_ref)
    y2_ref[...] = _dotT(h1, wl_ref[...])
    z2_ref[...] = _dotT(h1, wr_ref[...]) + b_ref[...]


def _post_body(s_ref, degp_ref, z_ref, g_ref, be_ref, out_ref):
    out_ref[...] = _bn_relu(s_ref, degp_ref, z_ref, g_ref, be_ref)


def kernel(x, edge_index, Wl1, Wr1, b1, g1, be1, Wl2, Wr2, b2, g2, be2):
    N, D = x.shape
    E = edge_index.shape[1]
    Np = (N + 16 * 8 - 1) // (16 * 8) * (16 * 8)
    NW = _NC * _NS
    ept = (E // NW + _K - 1) // _K * _K
    pad = NW * ept - E
    pidx = jnp.arange(pad, dtype=jnp.int32)
    src1 = jnp.concatenate([edge_index[0], pidx % N])
    dst1 = jnp.concatenate([edge_index[1], N + pidx % (Np - N)])
    ei = jnp.stack([src1.reshape(-1, _K), dst1.reshape(-1, _K)],
                   axis=1).reshape(-1)
    zeros = jnp.zeros((Np, D), jnp.float32)
    ones = jnp.ones((_K, D), jnp.float32)
    f32 = jnp.float32
    sd = jax.ShapeDtypeStruct

    degp = _sc_degree(ei, zeros, ones)

    y1, z1 = pl.pallas_call(
        _pre_body,
        out_shape=[sd((N, D), f32), sd((N, D), f32)],
    )(x, Wl1, Wr1, b1.reshape(1, D))

    s1 = _sc_scatter(y1, ei, zeros)

    y2, z2 = pl.pallas_call(
        _mid_body,
        out_shape=[sd((N, D), f32), sd((N, D), f32)],
    )(s1, degp, z1, g1.reshape(1, D), be1.reshape(1, D), Wl2, Wr2,
      b2.reshape(1, D))

    s2 = _sc_scatter(y2, ei, zeros)

    out = pl.pallas_call(
        _post_body,
        out_shape=sd((N, D), f32),
    )(s2, degp, z2, g2.reshape(1, D), be2.reshape(1, D))

    return out
